# baseline (speedup 1.0000x reference)
import dataclasses
import functools

import jax
import jax.numpy as jnp
from jax import lax
from jax.experimental import pallas as pl
from jax.experimental.pallas import tpu as pltpu
from jax.experimental.pallas import tpu_sc as plsc

_P = 640
_CHS = 400
_USE_SC_COUNTS = False


def _sc_counts(batch, P):
    (N,) = batch.shape
    nch = N // _CHS
    per_tile = -(-nch // 32)

    mesh = plsc.VectorSubcoreMesh(core_axis_name="c", subcore_axis_name="s")
    cp = pltpu.CompilerParams()
    if "needs_layout_passes" in pltpu.CompilerParams.__dataclass_fields__:
        cp = dataclasses.replace(cp, needs_layout_passes=False)

    @functools.partial(
        pl.kernel,
        out_type=jax.ShapeDtypeStruct((32, P, 16), jnp.float32),
        mesh=mesh,
        compiler_params=cp,
        scratch_types=[
            pltpu.VMEM((P, 16), jnp.float32),
            pltpu.VMEM((_CHS,), jnp.int32),
        ],
    )
    def hist(b_hbm, out_hbm, hist_v, chunk_v):
        cid = lax.axis_index("c")
        sid = lax.axis_index("s")
        wid = sid * 2 + cid
        zeros16 = jnp.zeros((16,), jnp.float32)
        ones16 = jnp.ones((16,), jnp.float32)
        lanes16 = lax.iota(jnp.int32, 16)

        @pl.loop(0, P)
        def _(g):
            hist_v[g, :] = zeros16

        @pl.loop(0, per_tile)
        def _(i):
            j = i * 32 + wid

            @pl.when(j < nch)
            def _():
                pltpu.sync_copy(b_hbm.at[pl.ds(j * _CHS, _CHS)], chunk_v)

                @pl.loop(0, _CHS, step=16)
                def _(r):
                    iv = chunk_v[pl.ds(r, 16)]
                    plsc.addupdate_scatter(hist_v, [iv, lanes16], ones16)

        pltpu.sync_copy(hist_v, out_hbm.at[wid])

    return hist(batch)


def _phase_a_body(batch_ref, h_ref, sums_ref, counts_ref, *, G):
    i = pl.program_id(0)
    b = batch_ref[0]
    B = b.shape[1]
    gids = jax.lax.broadcasted_iota(jnp.int32, (G, B), 0)
    oh_t = (gids == jnp.broadcast_to(b, (G, B)))
    oh_bf = oh_t.astype(jnp.bfloat16)
    h = h_ref[...]
    dn = (((1,), (0,)), ((), ()))
    part = jax.lax.dot_general(oh_bf, h.astype(jnp.bfloat16), dn,
                               preferred_element_type=jnp.float32)
    cnt = jnp.sum(oh_t.astype(jnp.float32), axis=1, keepdims=True)

    @pl.when(i == 0)
    def _():
        sums_ref[...] = part
        counts_ref[...] = cnt

    @pl.when(i != 0)
    def _():
        sums_ref[...] += part
        counts_ref[...] += cnt


def _phase_a_body_nocnt(batch_ref, h_ref, sums_ref, *, G):
    i = pl.program_id(0)
    b = batch_ref[0]
    B = b.shape[1]
    gids = jax.lax.broadcasted_iota(jnp.int32, (G, B), 0)
    oh_bf = (gids == jnp.broadcast_to(b, (G, B))).astype(jnp.bfloat16)
    dn = (((1,), (0,)), ((), ()))
    part = jax.lax.dot_general(oh_bf, h_ref[...].astype(jnp.bfloat16), dn,
                               preferred_element_type=jnp.float32)

    @pl.when(i == 0)
    def _():
        sums_ref[...] = part

    @pl.when(i != 0)
    def _():
        sums_ref[...] += part


def _mlp(x0, w1_ref, b1_ref, gamma_ref, beta_ref, w2_ref, b2_ref):
    dn_t = (((1,), (1,)), ((), ()))
    x = jax.lax.dot_general(x0, w1_ref[...], dn_t,
                            preferred_element_type=jnp.float32) + b1_ref[...]
    mu = jnp.mean(x, axis=-1, keepdims=True)
    var = jnp.mean((x - mu) ** 2, axis=-1, keepdims=True)
    x = (x - mu) * jax.lax.rsqrt(var + 1e-5) * gamma_ref[...] + beta_ref[...]
    x = jnp.maximum(x, 0.0)
    return jax.lax.dot_general(x, w2_ref[...], dn_t,
                               preferred_element_type=jnp.float32) + b2_ref[...]


def _phase_b_body(sums_ref, counts_ref, vn_h_ref, w1_ref, b1_ref, gamma_ref,
                  beta_ref, w2_ref, b2_ref, vn_out_ref, vn_hi_ref):
    mean = sums_ref[...] / jnp.maximum(counts_ref[...], 1.0)
    vn_out = _mlp(mean + vn_h_ref[...], w1_ref, b1_ref, gamma_ref, beta_ref,
                  w2_ref, b2_ref)
    vn_out_ref[...] = vn_out
    vn_hi_ref[...] = vn_out.astype(jnp.bfloat16)


def _phase_b_body_schist(sums_ref, hist_ref, vn_h_ref, w1_ref, b1_ref,
                         gamma_ref, beta_ref, w2_ref, b2_ref, vn_out_ref,
                         vn_hi_ref, *, G):
    hist = jnp.sum(hist_ref[...], axis=0)
    counts = jnp.sum(hist, axis=1, keepdims=True)[:G, :]
    mean = sums_ref[...] / jnp.maximum(counts, 1.0)
    vn_out = _mlp(mean + vn_h_ref[...], w1_ref, b1_ref, gamma_ref, beta_ref,
                  w2_ref, b2_ref)
    vn_out_ref[...] = vn_out
    vn_hi_ref[...] = vn_out.astype(jnp.bfloat16)


def _phase_c_body(batch_ref, h_ref, vn_hi_ref, out_ref, *, G):
    b = batch_ref[0]
    B = b.shape[1]
    gids = jax.lax.broadcasted_iota(jnp.int32, (G, B), 0)
    oh_bf = (gids == jnp.broadcast_to(b, (G, B))).astype(jnp.bfloat16)
    dn = (((0,), (0,)), ((), ()))
    g = jax.lax.dot_general(oh_bf, vn_hi_ref[...], dn,
                            preferred_element_type=jnp.float32)
    out_ref[...] = h_ref[...] + g


def _fused_body(batch_ref, h_ref, vn_h_ref, w1_ref, b1_ref, gamma_ref,
                beta_ref, w2_ref, b2_ref, out_ref, vn_out_ref,
                h_sc, sums_sc, counts_sc, vn_hi_sc, *, G, B, NB):
    i = pl.program_id(0)

    @pl.when(i < NB)
    def _():
        b = batch_ref[pl.ds(i, 1), :]
        gids = jax.lax.broadcasted_iota(jnp.int32, (G, B), 0)
        oh_t = (gids == jnp.broadcast_to(b, (G, B)))
        oh_bf = oh_t.astype(jnp.bfloat16)
        hblk = h_ref[...]
        h_sc[pl.ds(i * B, B), :] = hblk
        dn = (((1,), (0,)), ((), ()))
        part = jax.lax.dot_general(oh_bf, hblk.astype(jnp.bfloat16), dn,
                                   preferred_element_type=jnp.float32)
        cnt = jnp.sum(oh_t.astype(jnp.float32), axis=1, keepdims=True)

        @pl.when(i == 0)
        def _():
            sums_sc[...] = part
            counts_sc[...] = cnt

        @pl.when(i != 0)
        def _():
            sums_sc[...] += part
            counts_sc[...] += cnt

    @pl.when(i == NB)
    def _():
        mean = sums_sc[...] / jnp.maximum(counts_sc[...], 1.0)
        vn_out = _mlp(mean + vn_h_ref[...], w1_ref, b1_ref, gamma_ref,
                      beta_ref, w2_ref, b2_ref)
        vn_out_ref[...] = vn_out
        vn_hi_sc[...] = vn_out.astype(jnp.bfloat16)

    @pl.when(i > NB)
    def _():
        j = i - NB - 1
        b = batch_ref[pl.ds(j, 1), :]
        gids = jax.lax.broadcasted_iota(jnp.int32, (G, B), 0)
        oh_bf = (gids == jnp.broadcast_to(b, (G, B))).astype(jnp.bfloat16)
        dn = (((0,), (0,)), ((), ()))
        g = jax.lax.dot_general(oh_bf, vn_hi_sc[...], dn,
                                preferred_element_type=jnp.float32)
        out_ref[...] = h_sc[pl.ds(j * B, B), :] + g


def _pick_block(n):
    for cand in range(10240, 7, -8):
        if n % cand == 0:
            return cand
    return n


_FUSED = True


def kernel(h, batch, vn_h, W1, b1, gamma, beta, W2, b2, layer_idx):
    del layer_idx
    N, D = h.shape
    G = vn_h.shape[0]

    if _FUSED:
        B = 1000 if N % 1000 == 0 else _pick_block(N)
        NB = N // B
        batch2 = batch.astype(jnp.int32).reshape(NB, B)
        h_out, vn_out = pl.pallas_call(
            functools.partial(_fused_body, G=G, B=B, NB=NB),
            grid=(2 * NB + 1,),
            in_specs=[
                pl.BlockSpec((NB, B), lambda i: (0, 0)),
                pl.BlockSpec((B, D), lambda i: (jnp.minimum(i, NB - 1), 0)),
                pl.BlockSpec((G, D), lambda i: (0, 0)),
                pl.BlockSpec((D, D), lambda i: (0, 0)),
                pl.BlockSpec((1, D), lambda i: (0, 0)),
                pl.BlockSpec((1, D), lambda i: (0, 0)),
                pl.BlockSpec((1, D), lambda i: (0, 0)),
                pl.BlockSpec((D, D), lambda i: (0, 0)),
                pl.BlockSpec((1, D), lambda i: (0, 0)),
            ],
            out_specs=[
                pl.BlockSpec((B, D),
                             lambda i: (jnp.maximum(i - NB - 1, 0), 0)),
                pl.BlockSpec((G, D), lambda i: (0, 0)),
            ],
            out_shape=[
                jax.ShapeDtypeStruct((N, D), jnp.float32),
                jax.ShapeDtypeStruct((G, D), jnp.float32),
            ],
            scratch_shapes=[
                pltpu.VMEM((N, D), jnp.float32),
                pltpu.VMEM((G, D), jnp.float32),
                pltpu.VMEM((G, 1), jnp.float32),
                pltpu.VMEM((G, D), jnp.bfloat16),
            ],
        )(batch2, h, vn_h, W1, b1.reshape(1, D), gamma.reshape(1, D),
          beta.reshape(1, D), W2, b2.reshape(1, D))
        return (h_out, vn_out)

    B = _pick_block(N)
    NB = N // B
    batch_i = batch.astype(jnp.int32)
    batch3 = batch_i.reshape(NB, 1, B)

    mlp_args = (vn_h, W1, b1.reshape(1, D), gamma.reshape(1, D),
                beta.reshape(1, D), W2, b2.reshape(1, D))
    vn_shapes = [
        jax.ShapeDtypeStruct((G, D), jnp.float32),
        jax.ShapeDtypeStruct((G, D), jnp.bfloat16),
    ]

    if _USE_SC_COUNTS:
        hist = _sc_counts(batch_i, _P)
        sums = pl.pallas_call(
            functools.partial(_phase_a_body_nocnt, G=G),
            grid=(NB,),
            in_specs=[
                pl.BlockSpec((1, 1, B), lambda i: (i, 0, 0)),
                pl.BlockSpec((B, D), lambda i: (i, 0)),
            ],
            out_specs=pl.BlockSpec((G, D), lambda i: (0, 0)),
            out_shape=jax.ShapeDtypeStruct((G, D), jnp.float32),
        )(batch3, h)
        vn_out, vn_hi = pl.pallas_call(
            functools.partial(_phase_b_body_schist, G=G),
            out_shape=vn_shapes,
        )(sums, hist, *mlp_args)
    else:
        sums, counts = pl.pallas_call(
            functools.partial(_phase_a_body, G=G),
            grid=(NB,),
            in_specs=[
                pl.BlockSpec((1, 1, B), lambda i: (i, 0, 0)),
                pl.BlockSpec((B, D), lambda i: (i, 0)),
            ],
            out_specs=[
                pl.BlockSpec((G, D), lambda i: (0, 0)),
                pl.BlockSpec((G, 1), lambda i: (0, 0)),
            ],
            out_shape=[
                jax.ShapeDtypeStruct((G, D), jnp.float32),
                jax.ShapeDtypeStruct((G, 1), jnp.float32),
            ],
        )(batch3, h)
        vn_out, vn_hi = pl.pallas_call(
            _phase_b_body,
            out_shape=vn_shapes,
        )(sums, counts, *mlp_args)

    h_out = pl.pallas_call(
        functools.partial(_phase_c_body, G=G),
        grid=(NB,),
        in_specs=[
            pl.BlockSpec((1, 1, B), lambda i: (i, 0, 0)),
            pl.BlockSpec((B, D), lambda i: (i, 0)),
            pl.BlockSpec((G, D), lambda i: (0, 0)),
        ],
        out_specs=pl.BlockSpec((B, D), lambda i: (i, 0)),
        out_shape=jax.ShapeDtypeStruct((N, D), jnp.float32),
    )(batch3, h, vn_hi)

    return (h_out, vn_out)


# speedup vs baseline: 1.4200x; 1.4200x over previous
import dataclasses
import functools

import jax
import jax.numpy as jnp
from jax import lax
from jax.experimental import pallas as pl
from jax.experimental.pallas import tpu as pltpu
from jax.experimental.pallas import tpu_sc as plsc

_P = 640
_CHS = 400
_USE_SC_COUNTS = False


def _sc_counts(batch, P):
    (N,) = batch.shape
    nch = N // _CHS
    per_tile = -(-nch // 32)

    mesh = plsc.VectorSubcoreMesh(core_axis_name="c", subcore_axis_name="s")
    cp = pltpu.CompilerParams()
    if "needs_layout_passes" in pltpu.CompilerParams.__dataclass_fields__:
        cp = dataclasses.replace(cp, needs_layout_passes=False)

    @functools.partial(
        pl.kernel,
        out_type=jax.ShapeDtypeStruct((32, P, 16), jnp.float32),
        mesh=mesh,
        compiler_params=cp,
        scratch_types=[
            pltpu.VMEM((P, 16), jnp.float32),
            pltpu.VMEM((_CHS,), jnp.int32),
        ],
    )
    def hist(b_hbm, out_hbm, hist_v, chunk_v):
        cid = lax.axis_index("c")
        sid = lax.axis_index("s")
        wid = sid * 2 + cid
        zeros16 = jnp.zeros((16,), jnp.float32)
        ones16 = jnp.ones((16,), jnp.float32)
        lanes16 = lax.iota(jnp.int32, 16)

        @pl.loop(0, P)
        def _(g):
            hist_v[g, :] = zeros16

        @pl.loop(0, per_tile)
        def _(i):
            j = i * 32 + wid

            @pl.when(j < nch)
            def _():
                pltpu.sync_copy(b_hbm.at[pl.ds(j * _CHS, _CHS)], chunk_v)

                @pl.loop(0, _CHS, step=16)
                def _(r):
                    iv = chunk_v[pl.ds(r, 16)]
                    plsc.addupdate_scatter(hist_v, [iv, lanes16], ones16)

        pltpu.sync_copy(hist_v, out_hbm.at[wid])

    return hist(batch)


def _phase_a_body(batch_ref, h_ref, sums_ref, counts_ref, *, G):
    i = pl.program_id(0)
    b = batch_ref[0]
    B = b.shape[1]
    gids = jax.lax.broadcasted_iota(jnp.int32, (G, B), 0)
    oh_t = (gids == jnp.broadcast_to(b, (G, B)))
    oh_bf = oh_t.astype(jnp.bfloat16)
    h = h_ref[...]
    dn = (((1,), (0,)), ((), ()))
    part = jax.lax.dot_general(oh_bf, h.astype(jnp.bfloat16), dn,
                               preferred_element_type=jnp.float32)
    cnt = jnp.sum(oh_t.astype(jnp.float32), axis=1, keepdims=True)

    @pl.when(i == 0)
    def _():
        sums_ref[...] = part
        counts_ref[...] = cnt

    @pl.when(i != 0)
    def _():
        sums_ref[...] += part
        counts_ref[...] += cnt


def _phase_a_body_nocnt(batch_ref, h_ref, sums_ref, *, G):
    i = pl.program_id(0)
    b = batch_ref[0]
    B = b.shape[1]
    gids = jax.lax.broadcasted_iota(jnp.int32, (G, B), 0)
    oh_bf = (gids == jnp.broadcast_to(b, (G, B))).astype(jnp.bfloat16)
    dn = (((1,), (0,)), ((), ()))
    part = jax.lax.dot_general(oh_bf, h_ref[...].astype(jnp.bfloat16), dn,
                               preferred_element_type=jnp.float32)

    @pl.when(i == 0)
    def _():
        sums_ref[...] = part

    @pl.when(i != 0)
    def _():
        sums_ref[...] += part


def _mlp(x0, w1_ref, b1_ref, gamma_ref, beta_ref, w2_ref, b2_ref):
    dn_t = (((1,), (1,)), ((), ()))
    x = jax.lax.dot_general(x0, w1_ref[...], dn_t,
                            preferred_element_type=jnp.float32) + b1_ref[...]
    mu = jnp.mean(x, axis=-1, keepdims=True)
    var = jnp.mean((x - mu) ** 2, axis=-1, keepdims=True)
    x = (x - mu) * jax.lax.rsqrt(var + 1e-5) * gamma_ref[...] + beta_ref[...]
    x = jnp.maximum(x, 0.0)
    return jax.lax.dot_general(x, w2_ref[...], dn_t,
                               preferred_element_type=jnp.float32) + b2_ref[...]


def _phase_b_body(sums_ref, counts_ref, vn_h_ref, w1_ref, b1_ref, gamma_ref,
                  beta_ref, w2_ref, b2_ref, vn_out_ref, vn_hi_ref):
    mean = sums_ref[...] / jnp.maximum(counts_ref[...], 1.0)
    vn_out = _mlp(mean + vn_h_ref[...], w1_ref, b1_ref, gamma_ref, beta_ref,
                  w2_ref, b2_ref)
    vn_out_ref[...] = vn_out
    vn_hi_ref[...] = vn_out.astype(jnp.bfloat16)


def _phase_b_body_schist(sums_ref, hist_ref, vn_h_ref, w1_ref, b1_ref,
                         gamma_ref, beta_ref, w2_ref, b2_ref, vn_out_ref,
                         vn_hi_ref, *, G):
    hist = jnp.sum(hist_ref[...], axis=0)
    counts = jnp.sum(hist, axis=1, keepdims=True)[:G, :]
    mean = sums_ref[...] / jnp.maximum(counts, 1.0)
    vn_out = _mlp(mean + vn_h_ref[...], w1_ref, b1_ref, gamma_ref, beta_ref,
                  w2_ref, b2_ref)
    vn_out_ref[...] = vn_out
    vn_hi_ref[...] = vn_out.astype(jnp.bfloat16)


def _phase_c_body(batch_ref, h_ref, vn_hi_ref, out_ref, *, G):
    b = batch_ref[0]
    B = b.shape[1]
    gids = jax.lax.broadcasted_iota(jnp.int32, (G, B), 0)
    oh_bf = (gids == jnp.broadcast_to(b, (G, B))).astype(jnp.bfloat16)
    dn = (((0,), (0,)), ((), ()))
    g = jax.lax.dot_general(oh_bf, vn_hi_ref[...], dn,
                            preferred_element_type=jnp.float32)
    out_ref[...] = h_ref[...] + g


def _fused_body(batch_ref, h_ref, vn_h_ref, w1_ref, b1_ref, gamma_ref,
                beta_ref, w2_ref, b2_ref, out_ref, vn_out_ref,
                h_sc, sums_sc, counts_sc, vn_hi_sc, *, G, B, NB):
    i = pl.program_id(0)

    @pl.when(i < NB)
    def _():
        b = batch_ref[pl.ds(i, 1), :]
        gids = jax.lax.broadcasted_iota(jnp.int32, (G, B), 0)
        oh_t = (gids == jnp.broadcast_to(b, (G, B)))
        oh_bf = oh_t.astype(jnp.bfloat16)
        hblk = h_ref[...].astype(jnp.bfloat16)
        h_sc[pl.ds(i * B, B), :] = hblk
        dn = (((1,), (0,)), ((), ()))
        part = jax.lax.dot_general(oh_bf, hblk, dn,
                                   preferred_element_type=jnp.float32)
        cnt = jnp.sum(oh_t.astype(jnp.float32), axis=1, keepdims=True)

        @pl.when(i == 0)
        def _():
            sums_sc[...] = part
            counts_sc[...] = cnt

        @pl.when(i != 0)
        def _():
            sums_sc[...] += part
            counts_sc[...] += cnt

    @pl.when(i == NB)
    def _():
        mean = sums_sc[...] / jnp.maximum(counts_sc[...], 1.0)
        vn_out = _mlp(mean + vn_h_ref[...], w1_ref, b1_ref, gamma_ref,
                      beta_ref, w2_ref, b2_ref)
        vn_out_ref[...] = vn_out
        vn_hi_sc[...] = vn_out.astype(jnp.bfloat16)

    @pl.when(i > NB)
    def _():
        j = i - NB - 1
        b = batch_ref[pl.ds(j, 1), :]
        gids = jax.lax.broadcasted_iota(jnp.int32, (G, B), 0)
        oh_bf = (gids == jnp.broadcast_to(b, (G, B))).astype(jnp.bfloat16)
        dn = (((0,), (0,)), ((), ()))
        g = jax.lax.dot_general(oh_bf, vn_hi_sc[...], dn,
                                preferred_element_type=jnp.float32)
        out_ref[...] = h_sc[pl.ds(j * B, B), :].astype(jnp.float32) + g


def _pick_block(n):
    for cand in range(10240, 7, -8):
        if n % cand == 0:
            return cand
    return n


_FUSED = True


def kernel(h, batch, vn_h, W1, b1, gamma, beta, W2, b2, layer_idx):
    del layer_idx
    N, D = h.shape
    G = vn_h.shape[0]

    if _FUSED:
        B = 2000 if N % 2000 == 0 else _pick_block(N)
        NB = N // B
        batch2 = batch.astype(jnp.int32).reshape(NB, B)
        h_out, vn_out = pl.pallas_call(
            functools.partial(_fused_body, G=G, B=B, NB=NB),
            grid=(2 * NB + 1,),
            in_specs=[
                pl.BlockSpec((NB, B), lambda i: (0, 0)),
                pl.BlockSpec((B, D), lambda i: (jnp.minimum(i, NB - 1), 0)),
                pl.BlockSpec((G, D), lambda i: (0, 0)),
                pl.BlockSpec((D, D), lambda i: (0, 0)),
                pl.BlockSpec((1, D), lambda i: (0, 0)),
                pl.BlockSpec((1, D), lambda i: (0, 0)),
                pl.BlockSpec((1, D), lambda i: (0, 0)),
                pl.BlockSpec((D, D), lambda i: (0, 0)),
                pl.BlockSpec((1, D), lambda i: (0, 0)),
            ],
            out_specs=[
                pl.BlockSpec((B, D),
                             lambda i: (jnp.maximum(i - NB - 1, 0), 0)),
                pl.BlockSpec((G, D), lambda i: (0, 0)),
            ],
            out_shape=[
                jax.ShapeDtypeStruct((N, D), jnp.float32),
                jax.ShapeDtypeStruct((G, D), jnp.float32),
            ],
            scratch_shapes=[
                pltpu.VMEM((N, D), jnp.bfloat16),
                pltpu.VMEM((G, D), jnp.float32),
                pltpu.VMEM((G, 1), jnp.float32),
                pltpu.VMEM((G, D), jnp.bfloat16),
            ],
        )(batch2, h, vn_h, W1, b1.reshape(1, D), gamma.reshape(1, D),
          beta.reshape(1, D), W2, b2.reshape(1, D))
        return (h_out, vn_out)

    B = _pick_block(N)
    NB = N // B
    batch_i = batch.astype(jnp.int32)
    batch3 = batch_i.reshape(NB, 1, B)

    mlp_args = (vn_h, W1, b1.reshape(1, D), gamma.reshape(1, D),
                beta.reshape(1, D), W2, b2.reshape(1, D))
    vn_shapes = [
        jax.ShapeDtypeStruct((G, D), jnp.float32),
        jax.ShapeDtypeStruct((G, D), jnp.bfloat16),
    ]

    if _USE_SC_COUNTS:
        hist = _sc_counts(batch_i, _P)
        sums = pl.pallas_call(
            functools.partial(_phase_a_body_nocnt, G=G),
            grid=(NB,),
            in_specs=[
                pl.BlockSpec((1, 1, B), lambda i: (i, 0, 0)),
                pl.BlockSpec((B, D), lambda i: (i, 0)),
            ],
            out_specs=pl.BlockSpec((G, D), lambda i: (0, 0)),
            out_shape=jax.ShapeDtypeStruct((G, D), jnp.float32),
        )(batch3, h)
        vn_out, vn_hi = pl.pallas_call(
            functools.partial(_phase_b_body_schist, G=G),
            out_shape=vn_shapes,
        )(sums, hist, *mlp_args)
    else:
        sums, counts = pl.pallas_call(
            functools.partial(_phase_a_body, G=G),
            grid=(NB,),
            in_specs=[
                pl.BlockSpec((1, 1, B), lambda i: (i, 0, 0)),
                pl.BlockSpec((B, D), lambda i: (i, 0)),
            ],
            out_specs=[
                pl.BlockSpec((G, D), lambda i: (0, 0)),
                pl.BlockSpec((G, 1), lambda i: (0, 0)),
            ],
            out_shape=[
                jax.ShapeDtypeStruct((G, D), jnp.float32),
                jax.ShapeDtypeStruct((G, 1), jnp.float32),
            ],
        )(batch3, h)
        vn_out, vn_hi = pl.pallas_call(
            _phase_b_body,
            out_shape=vn_shapes,
        )(sums, counts, *mlp_args)

    h_out = pl.pallas_call(
        functools.partial(_phase_c_body, G=G),
        grid=(NB,),
        in_specs=[
            pl.BlockSpec((1, 1, B), lambda i: (i, 0, 0)),
            pl.BlockSpec((B, D), lambda i: (i, 0)),
            pl.BlockSpec((G, D), lambda i: (0, 0)),
        ],
        out_specs=pl.BlockSpec((B, D), lambda i: (i, 0)),
        out_shape=jax.ShapeDtypeStruct((N, D), jnp.float32),
    )(batch3, h, vn_hi)

    return (h_out, vn_out)


# speedup vs baseline: 1.8237x; 1.2843x over previous
import dataclasses
import functools

import jax
import jax.numpy as jnp
from jax import lax
from jax.experimental import pallas as pl
from jax.experimental.pallas import tpu as pltpu
from jax.experimental.pallas import tpu_sc as plsc

_P = 640
_CHS = 400
_USE_SC_COUNTS = False


def _sc_counts(batch, P):
    (N,) = batch.shape
    nch = N // _CHS
    per_tile = -(-nch // 32)

    mesh = plsc.VectorSubcoreMesh(core_axis_name="c", subcore_axis_name="s")
    cp = pltpu.CompilerParams()
    if "needs_layout_passes" in pltpu.CompilerParams.__dataclass_fields__:
        cp = dataclasses.replace(cp, needs_layout_passes=False)

    @functools.partial(
        pl.kernel,
        out_type=jax.ShapeDtypeStruct((32, P, 16), jnp.float32),
        mesh=mesh,
        compiler_params=cp,
        scratch_types=[
            pltpu.VMEM((P, 16), jnp.float32),
            pltpu.VMEM((_CHS,), jnp.int32),
        ],
    )
    def hist(b_hbm, out_hbm, hist_v, chunk_v):
        cid = lax.axis_index("c")
        sid = lax.axis_index("s")
        wid = sid * 2 + cid
        zeros16 = jnp.zeros((16,), jnp.float32)
        ones16 = jnp.ones((16,), jnp.float32)
        lanes16 = lax.iota(jnp.int32, 16)

        @pl.loop(0, P)
        def _(g):
            hist_v[g, :] = zeros16

        @pl.loop(0, per_tile)
        def _(i):
            j = i * 32 + wid

            @pl.when(j < nch)
            def _():
                pltpu.sync_copy(b_hbm.at[pl.ds(j * _CHS, _CHS)], chunk_v)

                @pl.loop(0, _CHS, step=16)
                def _(r):
                    iv = chunk_v[pl.ds(r, 16)]
                    plsc.addupdate_scatter(hist_v, [iv, lanes16], ones16)

        pltpu.sync_copy(hist_v, out_hbm.at[wid])

    return hist(batch)


def _phase_a_body(batch_ref, h_ref, sums_ref, counts_ref, *, G):
    i = pl.program_id(0)
    b = batch_ref[0]
    B = b.shape[1]
    gids = jax.lax.broadcasted_iota(jnp.int32, (G, B), 0)
    oh_t = (gids == jnp.broadcast_to(b, (G, B)))
    oh_bf = oh_t.astype(jnp.bfloat16)
    h = h_ref[...]
    dn = (((1,), (0,)), ((), ()))
    part = jax.lax.dot_general(oh_bf, h.astype(jnp.bfloat16), dn,
                               preferred_element_type=jnp.float32)
    cnt = jnp.sum(oh_t.astype(jnp.float32), axis=1, keepdims=True)

    @pl.when(i == 0)
    def _():
        sums_ref[...] = part
        counts_ref[...] = cnt

    @pl.when(i != 0)
    def _():
        sums_ref[...] += part
        counts_ref[...] += cnt


def _phase_a_body_nocnt(batch_ref, h_ref, sums_ref, *, G):
    i = pl.program_id(0)
    b = batch_ref[0]
    B = b.shape[1]
    gids = jax.lax.broadcasted_iota(jnp.int32, (G, B), 0)
    oh_bf = (gids == jnp.broadcast_to(b, (G, B))).astype(jnp.bfloat16)
    dn = (((1,), (0,)), ((), ()))
    part = jax.lax.dot_general(oh_bf, h_ref[...].astype(jnp.bfloat16), dn,
                               preferred_element_type=jnp.float32)

    @pl.when(i == 0)
    def _():
        sums_ref[...] = part

    @pl.when(i != 0)
    def _():
        sums_ref[...] += part


def _mlp(x0, w1_ref, b1_ref, gamma_ref, beta_ref, w2_ref, b2_ref):
    dn_t = (((1,), (1,)), ((), ()))
    x = jax.lax.dot_general(x0, w1_ref[...], dn_t,
                            preferred_element_type=jnp.float32) + b1_ref[...]
    mu = jnp.mean(x, axis=-1, keepdims=True)
    var = jnp.mean((x - mu) ** 2, axis=-1, keepdims=True)
    x = (x - mu) * jax.lax.rsqrt(var + 1e-5) * gamma_ref[...] + beta_ref[...]
    x = jnp.maximum(x, 0.0)
    return jax.lax.dot_general(x, w2_ref[...], dn_t,
                               preferred_element_type=jnp.float32) + b2_ref[...]


def _phase_b_body(sums_ref, counts_ref, vn_h_ref, w1_ref, b1_ref, gamma_ref,
                  beta_ref, w2_ref, b2_ref, vn_out_ref, vn_hi_ref):
    mean = sums_ref[...] / jnp.maximum(counts_ref[...], 1.0)
    vn_out = _mlp(mean + vn_h_ref[...], w1_ref, b1_ref, gamma_ref, beta_ref,
                  w2_ref, b2_ref)
    vn_out_ref[...] = vn_out
    vn_hi_ref[...] = vn_out.astype(jnp.bfloat16)


def _phase_b_body_schist(sums_ref, hist_ref, vn_h_ref, w1_ref, b1_ref,
                         gamma_ref, beta_ref, w2_ref, b2_ref, vn_out_ref,
                         vn_hi_ref, *, G):
    hist = jnp.sum(hist_ref[...], axis=0)
    counts = jnp.sum(hist, axis=1, keepdims=True)[:G, :]
    mean = sums_ref[...] / jnp.maximum(counts, 1.0)
    vn_out = _mlp(mean + vn_h_ref[...], w1_ref, b1_ref, gamma_ref, beta_ref,
                  w2_ref, b2_ref)
    vn_out_ref[...] = vn_out
    vn_hi_ref[...] = vn_out.astype(jnp.bfloat16)


def _phase_c_body(batch_ref, h_ref, vn_hi_ref, out_ref, *, G):
    b = batch_ref[0]
    B = b.shape[1]
    gids = jax.lax.broadcasted_iota(jnp.int32, (G, B), 0)
    oh_bf = (gids == jnp.broadcast_to(b, (G, B))).astype(jnp.bfloat16)
    dn = (((0,), (0,)), ((), ()))
    g = jax.lax.dot_general(oh_bf, vn_hi_ref[...], dn,
                            preferred_element_type=jnp.float32)
    out_ref[...] = h_ref[...] + g


def _fused_body(batch_ref, h_ref, vn_h_ref, w1_ref, b1_ref, gamma_ref,
                beta_ref, w2_ref, b2_ref, out_ref, vn_out_ref,
                h_sc, sums_sc, counts_sc, vn_hi_sc, *, G, B, NB):
    i = pl.program_id(0)

    @pl.when(i < NB)
    def _():
        b = batch_ref[pl.ds(i, 1), :]
        gids = jax.lax.broadcasted_iota(jnp.int32, (G, B), 0)
        oh_t = (gids == jnp.broadcast_to(b, (G, B)))
        oh_bf = oh_t.astype(jnp.bfloat16)
        hblk = h_ref[...].astype(jnp.bfloat16)
        h_sc[pl.ds(i * B, B), :] = hblk
        dn = (((1,), (0,)), ((), ()))
        part = jax.lax.dot_general(oh_bf, hblk, dn,
                                   preferred_element_type=jnp.float32)
        cnt = jnp.sum(oh_t.astype(jnp.float32), axis=1, keepdims=True)

        @pl.when(i == 0)
        def _():
            sums_sc[...] = part
            counts_sc[...] = cnt

        @pl.when(i != 0)
        def _():
            sums_sc[...] += part
            counts_sc[...] += cnt

    @pl.when(i == NB)
    def _():
        mean = sums_sc[...] / jnp.maximum(counts_sc[...], 1.0)
        vn_out = _mlp(mean + vn_h_ref[...], w1_ref, b1_ref, gamma_ref,
                      beta_ref, w2_ref, b2_ref)
        vn_out_ref[...] = vn_out
        vn_hi_sc[...] = vn_out.astype(jnp.bfloat16)

    @pl.when(i > NB)
    def _():
        j = i - NB - 1
        b = batch_ref[pl.ds(j, 1), :]
        gids = jax.lax.broadcasted_iota(jnp.int32, (G, B), 0)
        oh_bf = (gids == jnp.broadcast_to(b, (G, B))).astype(jnp.bfloat16)
        dn = (((0,), (0,)), ((), ()))
        g = jax.lax.dot_general(oh_bf, vn_hi_sc[...], dn,
                                preferred_element_type=jnp.float32)
        out_ref[...] = h_sc[pl.ds(j * B, B), :].astype(jnp.float32) + g


def _pick_block(n):
    for cand in range(10240, 7, -8):
        if n % cand == 0:
            return cand
    return n


_FUSED = True


def kernel(h, batch, vn_h, W1, b1, gamma, beta, W2, b2, layer_idx):
    del layer_idx
    N, D = h.shape
    G = vn_h.shape[0]

    if _FUSED:
        B = 5000 if N % 5000 == 0 else _pick_block(N)
        NB = N // B
        batch2 = batch.astype(jnp.int32).reshape(NB, B)
        h_out, vn_out = pl.pallas_call(
            functools.partial(_fused_body, G=G, B=B, NB=NB),
            grid=(2 * NB + 1,),
            in_specs=[
                pl.BlockSpec((NB, B), lambda i: (0, 0)),
                pl.BlockSpec((B, D), lambda i: (jnp.minimum(i, NB - 1), 0)),
                pl.BlockSpec((G, D), lambda i: (0, 0)),
                pl.BlockSpec((D, D), lambda i: (0, 0)),
                pl.BlockSpec((1, D), lambda i: (0, 0)),
                pl.BlockSpec((1, D), lambda i: (0, 0)),
                pl.BlockSpec((1, D), lambda i: (0, 0)),
                pl.BlockSpec((D, D), lambda i: (0, 0)),
                pl.BlockSpec((1, D), lambda i: (0, 0)),
            ],
            out_specs=[
                pl.BlockSpec((B, D),
                             lambda i: (jnp.maximum(i - NB - 1, 0), 0)),
                pl.BlockSpec((G, D), lambda i: (0, 0)),
            ],
            out_shape=[
                jax.ShapeDtypeStruct((N, D), jnp.float32),
                jax.ShapeDtypeStruct((G, D), jnp.float32),
            ],
            scratch_shapes=[
                pltpu.VMEM((N, D), jnp.bfloat16),
                pltpu.VMEM((G, D), jnp.float32),
                pltpu.VMEM((G, 1), jnp.float32),
                pltpu.VMEM((G, D), jnp.bfloat16),
            ],
        )(batch2, h, vn_h, W1, b1.reshape(1, D), gamma.reshape(1, D),
          beta.reshape(1, D), W2, b2.reshape(1, D))
        return (h_out, vn_out)

    B = _pick_block(N)
    NB = N // B
    batch_i = batch.astype(jnp.int32)
    batch3 = batch_i.reshape(NB, 1, B)

    mlp_args = (vn_h, W1, b1.reshape(1, D), gamma.reshape(1, D),
                beta.reshape(1, D), W2, b2.reshape(1, D))
    vn_shapes = [
        jax.ShapeDtypeStruct((G, D), jnp.float32),
        jax.ShapeDtypeStruct((G, D), jnp.bfloat16),
    ]

    if _USE_SC_COUNTS:
        hist = _sc_counts(batch_i, _P)
        sums = pl.pallas_call(
            functools.partial(_phase_a_body_nocnt, G=G),
            grid=(NB,),
            in_specs=[
                pl.BlockSpec((1, 1, B), lambda i: (i, 0, 0)),
                pl.BlockSpec((B, D), lambda i: (i, 0)),
            ],
            out_specs=pl.BlockSpec((G, D), lambda i: (0, 0)),
            out_shape=jax.ShapeDtypeStruct((G, D), jnp.float32),
        )(batch3, h)
        vn_out, vn_hi = pl.pallas_call(
            functools.partial(_phase_b_body_schist, G=G),
            out_shape=vn_shapes,
        )(sums, hist, *mlp_args)
    else:
        sums, counts = pl.pallas_call(
            functools.partial(_phase_a_body, G=G),
            grid=(NB,),
            in_specs=[
                pl.BlockSpec((1, 1, B), lambda i: (i, 0, 0)),
                pl.BlockSpec((B, D), lambda i: (i, 0)),
            ],
            out_specs=[
                pl.BlockSpec((G, D), lambda i: (0, 0)),
                pl.BlockSpec((G, 1), lambda i: (0, 0)),
            ],
            out_shape=[
                jax.ShapeDtypeStruct((G, D), jnp.float32),
                jax.ShapeDtypeStruct((G, 1), jnp.float32),
            ],
        )(batch3, h)
        vn_out, vn_hi = pl.pallas_call(
            _phase_b_body,
            out_shape=vn_shapes,
        )(sums, counts, *mlp_args)

    h_out = pl.pallas_call(
        functools.partial(_phase_c_body, G=G),
        grid=(NB,),
        in_specs=[
            pl.BlockSpec((1, 1, B), lambda i: (i, 0, 0)),
            pl.BlockSpec((B, D), lambda i: (i, 0)),
            pl.BlockSpec((G, D), lambda i: (0, 0)),
        ],
        out_specs=pl.BlockSpec((B, D), lambda i: (i, 0)),
        out_shape=jax.ShapeDtypeStruct((N, D), jnp.float32),
    )(batch3, h, vn_hi)

    return (h_out, vn_out)


# speedup vs baseline: 1.8447x; 1.0115x over previous
import dataclasses
import functools

import jax
import jax.numpy as jnp
from jax import lax
from jax.experimental import pallas as pl
from jax.experimental.pallas import tpu as pltpu
from jax.experimental.pallas import tpu_sc as plsc

_P = 640
_CHS = 400
_USE_SC_COUNTS = False


def _sc_counts(batch, P):
    (N,) = batch.shape
    nch = N // _CHS
    per_tile = -(-nch // 32)

    mesh = plsc.VectorSubcoreMesh(core_axis_name="c", subcore_axis_name="s")
    cp = pltpu.CompilerParams()
    if "needs_layout_passes" in pltpu.CompilerParams.__dataclass_fields__:
        cp = dataclasses.replace(cp, needs_layout_passes=False)

    @functools.partial(
        pl.kernel,
        out_type=jax.ShapeDtypeStruct((32, P, 16), jnp.float32),
        mesh=mesh,
        compiler_params=cp,
        scratch_types=[
            pltpu.VMEM((P, 16), jnp.float32),
            pltpu.VMEM((_CHS,), jnp.int32),
        ],
    )
    def hist(b_hbm, out_hbm, hist_v, chunk_v):
        cid = lax.axis_index("c")
        sid = lax.axis_index("s")
        wid = sid * 2 + cid
        zeros16 = jnp.zeros((16,), jnp.float32)
        ones16 = jnp.ones((16,), jnp.float32)
        lanes16 = lax.iota(jnp.int32, 16)

        @pl.loop(0, P)
        def _(g):
            hist_v[g, :] = zeros16

        @pl.loop(0, per_tile)
        def _(i):
            j = i * 32 + wid

            @pl.when(j < nch)
            def _():
                pltpu.sync_copy(b_hbm.at[pl.ds(j * _CHS, _CHS)], chunk_v)

                @pl.loop(0, _CHS, step=16)
                def _(r):
                    iv = chunk_v[pl.ds(r, 16)]
                    plsc.addupdate_scatter(hist_v, [iv, lanes16], ones16)

        pltpu.sync_copy(hist_v, out_hbm.at[wid])

    return hist(batch)


def _phase_a_body(batch_ref, h_ref, sums_ref, counts_ref, *, G):
    i = pl.program_id(0)
    b = batch_ref[0]
    B = b.shape[1]
    gids = jax.lax.broadcasted_iota(jnp.int32, (G, B), 0)
    oh_t = (gids == jnp.broadcast_to(b, (G, B)))
    oh_bf = oh_t.astype(jnp.bfloat16)
    h = h_ref[...]
    dn = (((1,), (0,)), ((), ()))
    part = jax.lax.dot_general(oh_bf, h.astype(jnp.bfloat16), dn,
                               preferred_element_type=jnp.float32)
    cnt = jnp.sum(oh_t.astype(jnp.float32), axis=1, keepdims=True)

    @pl.when(i == 0)
    def _():
        sums_ref[...] = part
        counts_ref[...] = cnt

    @pl.when(i != 0)
    def _():
        sums_ref[...] += part
        counts_ref[...] += cnt


def _phase_a_body_nocnt(batch_ref, h_ref, sums_ref, *, G):
    i = pl.program_id(0)
    b = batch_ref[0]
    B = b.shape[1]
    gids = jax.lax.broadcasted_iota(jnp.int32, (G, B), 0)
    oh_bf = (gids == jnp.broadcast_to(b, (G, B))).astype(jnp.bfloat16)
    dn = (((1,), (0,)), ((), ()))
    part = jax.lax.dot_general(oh_bf, h_ref[...].astype(jnp.bfloat16), dn,
                               preferred_element_type=jnp.float32)

    @pl.when(i == 0)
    def _():
        sums_ref[...] = part

    @pl.when(i != 0)
    def _():
        sums_ref[...] += part


def _mlp(x0, w1_ref, b1_ref, gamma_ref, beta_ref, w2_ref, b2_ref):
    dn_t = (((1,), (1,)), ((), ()))
    x = jax.lax.dot_general(x0, w1_ref[...], dn_t,
                            preferred_element_type=jnp.float32) + b1_ref[...]
    mu = jnp.mean(x, axis=-1, keepdims=True)
    var = jnp.mean((x - mu) ** 2, axis=-1, keepdims=True)
    x = (x - mu) * jax.lax.rsqrt(var + 1e-5) * gamma_ref[...] + beta_ref[...]
    x = jnp.maximum(x, 0.0)
    return jax.lax.dot_general(x, w2_ref[...], dn_t,
                               preferred_element_type=jnp.float32) + b2_ref[...]


def _phase_b_body(sums_ref, counts_ref, vn_h_ref, w1_ref, b1_ref, gamma_ref,
                  beta_ref, w2_ref, b2_ref, vn_out_ref, vn_hi_ref):
    mean = sums_ref[...] / jnp.maximum(counts_ref[...], 1.0)
    vn_out = _mlp(mean + vn_h_ref[...], w1_ref, b1_ref, gamma_ref, beta_ref,
                  w2_ref, b2_ref)
    vn_out_ref[...] = vn_out
    vn_hi_ref[...] = vn_out.astype(jnp.bfloat16)


def _phase_b_body_schist(sums_ref, hist_ref, vn_h_ref, w1_ref, b1_ref,
                         gamma_ref, beta_ref, w2_ref, b2_ref, vn_out_ref,
                         vn_hi_ref, *, G):
    hist = jnp.sum(hist_ref[...], axis=0)
    counts = jnp.sum(hist, axis=1, keepdims=True)[:G, :]
    mean = sums_ref[...] / jnp.maximum(counts, 1.0)
    vn_out = _mlp(mean + vn_h_ref[...], w1_ref, b1_ref, gamma_ref, beta_ref,
                  w2_ref, b2_ref)
    vn_out_ref[...] = vn_out
    vn_hi_ref[...] = vn_out.astype(jnp.bfloat16)


def _phase_c_body(batch_ref, h_ref, vn_hi_ref, out_ref, *, G):
    b = batch_ref[0]
    B = b.shape[1]
    gids = jax.lax.broadcasted_iota(jnp.int32, (G, B), 0)
    oh_bf = (gids == jnp.broadcast_to(b, (G, B))).astype(jnp.bfloat16)
    dn = (((0,), (0,)), ((), ()))
    g = jax.lax.dot_general(oh_bf, vn_hi_ref[...], dn,
                            preferred_element_type=jnp.float32)
    out_ref[...] = h_ref[...] + g


def _fused_body(batch_ref, h_ref, vn_h_ref, w1_ref, b1_ref, gamma_ref,
                beta_ref, w2_ref, b2_ref, out_ref, vn_out_ref,
                h_sc, sums_sc, counts_sc, vn_hi_sc, *, G, B, NB, GC):
    i = pl.program_id(0)
    nk = G // GC

    @pl.when(i < NB)
    def _():
        @pl.when(i == 0)
        def _():
            sums_sc[...] = jnp.zeros_like(sums_sc)
            counts_sc[...] = jnp.zeros_like(counts_sc)

        b = batch_ref[pl.ds(i, 1), :]
        g_lo = jnp.min(b)
        g_hi = jnp.max(b)
        hblk = h_ref[...].astype(jnp.bfloat16)
        h_sc[pl.ds(i * B, B), :] = hblk
        gids = jax.lax.broadcasted_iota(jnp.int32, (GC, B), 0)
        dn = (((1,), (0,)), ((), ()))
        for k in range(nk):
            @pl.when((g_lo < (k + 1) * GC) & (g_hi >= k * GC))
            def _(k=k):
                oh_t = (gids == jnp.broadcast_to(b - (k * GC), (GC, B)))
                oh_bf = oh_t.astype(jnp.bfloat16)
                part = jax.lax.dot_general(oh_bf, hblk, dn,
                                           preferred_element_type=jnp.float32)
                cnt = jnp.sum(oh_t.astype(jnp.float32), axis=1, keepdims=True)
                sums_sc[pl.ds(k * GC, GC), :] += part
                counts_sc[pl.ds(k * GC, GC), :] += cnt

    @pl.when(i == NB)
    def _():
        mean = sums_sc[...] / jnp.maximum(counts_sc[...], 1.0)
        vn_out = _mlp(mean + vn_h_ref[...], w1_ref, b1_ref, gamma_ref,
                      beta_ref, w2_ref, b2_ref)
        vn_out_ref[...] = vn_out
        vn_hi_sc[...] = vn_out.astype(jnp.bfloat16)

    @pl.when(i > NB)
    def _():
        j = i - NB - 1
        b = batch_ref[pl.ds(j, 1), :]
        g_lo = jnp.min(b)
        g_hi = jnp.max(b)
        out_ref[...] = h_sc[pl.ds(j * B, B), :].astype(jnp.float32)
        gids = jax.lax.broadcasted_iota(jnp.int32, (GC, B), 0)
        dn = (((0,), (0,)), ((), ()))
        for k in range(nk):
            @pl.when((g_lo < (k + 1) * GC) & (g_hi >= k * GC))
            def _(k=k):
                oh_bf = (gids == jnp.broadcast_to(b - (k * GC), (GC, B))
                         ).astype(jnp.bfloat16)
                g = jax.lax.dot_general(
                    oh_bf, vn_hi_sc[pl.ds(k * GC, GC), :], dn,
                    preferred_element_type=jnp.float32)
                out_ref[...] += g


def _pick_block(n):
    for cand in range(10240, 7, -8):
        if n % cand == 0:
            return cand
    return n


_FUSED = True


def kernel(h, batch, vn_h, W1, b1, gamma, beta, W2, b2, layer_idx):
    del layer_idx
    N, D = h.shape
    G = vn_h.shape[0]

    if _FUSED:
        B = 5000 if N % 5000 == 0 else _pick_block(N)
        NB = N // B
        batch2 = batch.astype(jnp.int32).reshape(NB, B)
        h_out, vn_out = pl.pallas_call(
            functools.partial(_fused_body, G=G, B=B, NB=NB, GC=128),
            grid=(2 * NB + 1,),
            in_specs=[
                pl.BlockSpec((NB, B), lambda i: (0, 0)),
                pl.BlockSpec((B, D), lambda i: (jnp.minimum(i, NB - 1), 0)),
                pl.BlockSpec((G, D), lambda i: (0, 0)),
                pl.BlockSpec((D, D), lambda i: (0, 0)),
                pl.BlockSpec((1, D), lambda i: (0, 0)),
                pl.BlockSpec((1, D), lambda i: (0, 0)),
                pl.BlockSpec((1, D), lambda i: (0, 0)),
                pl.BlockSpec((D, D), lambda i: (0, 0)),
                pl.BlockSpec((1, D), lambda i: (0, 0)),
            ],
            out_specs=[
                pl.BlockSpec((B, D),
                             lambda i: (jnp.maximum(i - NB - 1, 0), 0)),
                pl.BlockSpec((G, D), lambda i: (0, 0)),
            ],
            out_shape=[
                jax.ShapeDtypeStruct((N, D), jnp.float32),
                jax.ShapeDtypeStruct((G, D), jnp.float32),
            ],
            scratch_shapes=[
                pltpu.VMEM((N, D), jnp.bfloat16),
                pltpu.VMEM((G, D), jnp.float32),
                pltpu.VMEM((G, 1), jnp.float32),
                pltpu.VMEM((G, D), jnp.bfloat16),
            ],
        )(batch2, h, vn_h, W1, b1.reshape(1, D), gamma.reshape(1, D),
          beta.reshape(1, D), W2, b2.reshape(1, D))
        return (h_out, vn_out)

    B = _pick_block(N)
    NB = N // B
    batch_i = batch.astype(jnp.int32)
    batch3 = batch_i.reshape(NB, 1, B)

    mlp_args = (vn_h, W1, b1.reshape(1, D), gamma.reshape(1, D),
                beta.reshape(1, D), W2, b2.reshape(1, D))
    vn_shapes = [
        jax.ShapeDtypeStruct((G, D), jnp.float32),
        jax.ShapeDtypeStruct((G, D), jnp.bfloat16),
    ]

    if _USE_SC_COUNTS:
        hist = _sc_counts(batch_i, _P)
        sums = pl.pallas_call(
            functools.partial(_phase_a_body_nocnt, G=G),
            grid=(NB,),
            in_specs=[
                pl.BlockSpec((1, 1, B), lambda i: (i, 0, 0)),
                pl.BlockSpec((B, D), lambda i: (i, 0)),
            ],
            out_specs=pl.BlockSpec((G, D), lambda i: (0, 0)),
            out_shape=jax.ShapeDtypeStruct((G, D), jnp.float32),
        )(batch3, h)
        vn_out, vn_hi = pl.pallas_call(
            functools.partial(_phase_b_body_schist, G=G),
            out_shape=vn_shapes,
        )(sums, hist, *mlp_args)
    else:
        sums, counts = pl.pallas_call(
            functools.partial(_phase_a_body, G=G),
            grid=(NB,),
            in_specs=[
                pl.BlockSpec((1, 1, B), lambda i: (i, 0, 0)),
                pl.BlockSpec((B, D), lambda i: (i, 0)),
            ],
            out_specs=[
                pl.BlockSpec((G, D), lambda i: (0, 0)),
                pl.BlockSpec((G, 1), lambda i: (0, 0)),
            ],
            out_shape=[
                jax.ShapeDtypeStruct((G, D), jnp.float32),
                jax.ShapeDtypeStruct((G, 1), jnp.float32),
            ],
        )(batch3, h)
        vn_out, vn_hi = pl.pallas_call(
            _phase_b_body,
            out_shape=vn_shapes,
        )(sums, counts, *mlp_args)

    h_out = pl.pallas_call(
        functools.partial(_phase_c_body, G=G),
        grid=(NB,),
        in_specs=[
            pl.BlockSpec((1, 1, B), lambda i: (i, 0, 0)),
            pl.BlockSpec((B, D), lambda i: (i, 0)),
            pl.BlockSpec((G, D), lambda i: (0, 0)),
        ],
        out_specs=pl.BlockSpec((B, D), lambda i: (i, 0)),
        out_shape=jax.ShapeDtypeStruct((N, D), jnp.float32),
    )(batch3, h, vn_hi)

    return (h_out, vn_out)


# speedup vs baseline: 1.9490x; 1.0565x over previous
import dataclasses
import functools

import jax
import jax.numpy as jnp
from jax import lax
from jax.experimental import pallas as pl
from jax.experimental.pallas import tpu as pltpu
from jax.experimental.pallas import tpu_sc as plsc

_P = 640
_CHS = 400
_USE_SC_COUNTS = False


def _sc_counts(batch, P):
    (N,) = batch.shape
    nch = N // _CHS
    per_tile = -(-nch // 32)

    mesh = plsc.VectorSubcoreMesh(core_axis_name="c", subcore_axis_name="s")
    cp = pltpu.CompilerParams()
    if "needs_layout_passes" in pltpu.CompilerParams.__dataclass_fields__:
        cp = dataclasses.replace(cp, needs_layout_passes=False)

    @functools.partial(
        pl.kernel,
        out_type=jax.ShapeDtypeStruct((32, P, 16), jnp.float32),
        mesh=mesh,
        compiler_params=cp,
        scratch_types=[
            pltpu.VMEM((P, 16), jnp.float32),
            pltpu.VMEM((_CHS,), jnp.int32),
        ],
    )
    def hist(b_hbm, out_hbm, hist_v, chunk_v):
        cid = lax.axis_index("c")
        sid = lax.axis_index("s")
        wid = sid * 2 + cid
        zeros16 = jnp.zeros((16,), jnp.float32)
        ones16 = jnp.ones((16,), jnp.float32)
        lanes16 = lax.iota(jnp.int32, 16)

        @pl.loop(0, P)
        def _(g):
            hist_v[g, :] = zeros16

        @pl.loop(0, per_tile)
        def _(i):
            j = i * 32 + wid

            @pl.when(j < nch)
            def _():
                pltpu.sync_copy(b_hbm.at[pl.ds(j * _CHS, _CHS)], chunk_v)

                @pl.loop(0, _CHS, step=16)
                def _(r):
                    iv = chunk_v[pl.ds(r, 16)]
                    plsc.addupdate_scatter(hist_v, [iv, lanes16], ones16)

        pltpu.sync_copy(hist_v, out_hbm.at[wid])

    return hist(batch)


def _phase_a_body(batch_ref, h_ref, sums_ref, counts_ref, *, G):
    i = pl.program_id(0)
    b = batch_ref[0]
    B = b.shape[1]
    gids = jax.lax.broadcasted_iota(jnp.int32, (G, B), 0)
    oh_t = (gids == jnp.broadcast_to(b, (G, B)))
    oh_bf = oh_t.astype(jnp.bfloat16)
    h = h_ref[...]
    dn = (((1,), (0,)), ((), ()))
    part = jax.lax.dot_general(oh_bf, h.astype(jnp.bfloat16), dn,
                               preferred_element_type=jnp.float32)
    cnt = jnp.sum(oh_t.astype(jnp.float32), axis=1, keepdims=True)

    @pl.when(i == 0)
    def _():
        sums_ref[...] = part
        counts_ref[...] = cnt

    @pl.when(i != 0)
    def _():
        sums_ref[...] += part
        counts_ref[...] += cnt


def _phase_a_body_nocnt(batch_ref, h_ref, sums_ref, *, G):
    i = pl.program_id(0)
    b = batch_ref[0]
    B = b.shape[1]
    gids = jax.lax.broadcasted_iota(jnp.int32, (G, B), 0)
    oh_bf = (gids == jnp.broadcast_to(b, (G, B))).astype(jnp.bfloat16)
    dn = (((1,), (0,)), ((), ()))
    part = jax.lax.dot_general(oh_bf, h_ref[...].astype(jnp.bfloat16), dn,
                               preferred_element_type=jnp.float32)

    @pl.when(i == 0)
    def _():
        sums_ref[...] = part

    @pl.when(i != 0)
    def _():
        sums_ref[...] += part


def _mlp(x0, w1_ref, b1_ref, gamma_ref, beta_ref, w2_ref, b2_ref):
    dn_t = (((1,), (1,)), ((), ()))
    x = jax.lax.dot_general(x0, w1_ref[...], dn_t,
                            preferred_element_type=jnp.float32) + b1_ref[...]
    mu = jnp.mean(x, axis=-1, keepdims=True)
    var = jnp.mean((x - mu) ** 2, axis=-1, keepdims=True)
    x = (x - mu) * jax.lax.rsqrt(var + 1e-5) * gamma_ref[...] + beta_ref[...]
    x = jnp.maximum(x, 0.0)
    return jax.lax.dot_general(x, w2_ref[...], dn_t,
                               preferred_element_type=jnp.float32) + b2_ref[...]


def _phase_b_body(sums_ref, counts_ref, vn_h_ref, w1_ref, b1_ref, gamma_ref,
                  beta_ref, w2_ref, b2_ref, vn_out_ref, vn_hi_ref):
    mean = sums_ref[...] / jnp.maximum(counts_ref[...], 1.0)
    vn_out = _mlp(mean + vn_h_ref[...], w1_ref, b1_ref, gamma_ref, beta_ref,
                  w2_ref, b2_ref)
    vn_out_ref[...] = vn_out
    vn_hi_ref[...] = vn_out.astype(jnp.bfloat16)


def _phase_b_body_schist(sums_ref, hist_ref, vn_h_ref, w1_ref, b1_ref,
                         gamma_ref, beta_ref, w2_ref, b2_ref, vn_out_ref,
                         vn_hi_ref, *, G):
    hist = jnp.sum(hist_ref[...], axis=0)
    counts = jnp.sum(hist, axis=1, keepdims=True)[:G, :]
    mean = sums_ref[...] / jnp.maximum(counts, 1.0)
    vn_out = _mlp(mean + vn_h_ref[...], w1_ref, b1_ref, gamma_ref, beta_ref,
                  w2_ref, b2_ref)
    vn_out_ref[...] = vn_out
    vn_hi_ref[...] = vn_out.astype(jnp.bfloat16)


def _phase_c_body(batch_ref, h_ref, vn_hi_ref, out_ref, *, G):
    b = batch_ref[0]
    B = b.shape[1]
    gids = jax.lax.broadcasted_iota(jnp.int32, (G, B), 0)
    oh_bf = (gids == jnp.broadcast_to(b, (G, B))).astype(jnp.bfloat16)
    dn = (((0,), (0,)), ((), ()))
    g = jax.lax.dot_general(oh_bf, vn_hi_ref[...], dn,
                            preferred_element_type=jnp.float32)
    out_ref[...] = h_ref[...] + g


def _fused_body(batch_ref, h_ref, vn_h_ref, w1_ref, b1_ref, gamma_ref,
                beta_ref, w2_ref, b2_ref, out_ref, vn_out_ref,
                h_sc, sums_sc, counts_sc, vn_hi_sc, *, G, B, NB, GC):
    i = pl.program_id(0)
    nk = G // GC
    W = 2 * GC

    @pl.when(i < NB)
    def _():
        @pl.when(i == 0)
        def _():
            sums_sc[...] = jnp.zeros_like(sums_sc)
            counts_sc[...] = jnp.zeros_like(counts_sc)

        b = batch_ref[pl.ds(i, 1), :]
        g_lo = jnp.min(b)
        g_hi = jnp.max(b)
        g0 = (g_lo // GC) * GC
        hblk = h_ref[...].astype(jnp.bfloat16)
        h_sc[pl.ds(i * B, B), :] = hblk
        gids = jax.lax.broadcasted_iota(jnp.int32, (W, B), 0)
        dn = (((1,), (0,)), ((), ()))
        bw = jnp.broadcast_to(b - g0, (W, B))
        oh_t = (gids == bw)
        oh_bf = oh_t.astype(jnp.bfloat16)
        part = jax.lax.dot_general(oh_bf, hblk, dn,
                                   preferred_element_type=jnp.float32)
        cnt = jnp.sum(oh_t.astype(jnp.float32), axis=1, keepdims=True)
        sums_sc[pl.ds(g0, W), :] += part
        counts_sc[pl.ds(g0, W), :] += cnt
        gids_t = jax.lax.broadcasted_iota(jnp.int32, (GC, B), 0)
        for k in range(2, nk):
            @pl.when((g_hi >= k * GC) & (k * GC >= g0 + W))
            def _(k=k):
                oh_tk = (gids_t == jnp.broadcast_to(b - (k * GC), (GC, B)))
                oh_bk = oh_tk.astype(jnp.bfloat16)
                pk = jax.lax.dot_general(oh_bk, hblk, dn,
                                         preferred_element_type=jnp.float32)
                ck = jnp.sum(oh_tk.astype(jnp.float32), axis=1, keepdims=True)
                sums_sc[pl.ds(k * GC, GC), :] += pk
                counts_sc[pl.ds(k * GC, GC), :] += ck

    @pl.when(i == NB)
    def _():
        mean = (sums_sc[pl.ds(0, G), :]
                / jnp.maximum(counts_sc[pl.ds(0, G), :], 1.0))
        vn_out = _mlp(mean + vn_h_ref[...], w1_ref, b1_ref, gamma_ref,
                      beta_ref, w2_ref, b2_ref)
        vn_out_ref[...] = vn_out
        vn_hi_sc[pl.ds(0, G), :] = vn_out.astype(jnp.bfloat16)
        vn_hi_sc[pl.ds(G, GC), :] = jnp.zeros((GC, vn_out.shape[1]),
                                              jnp.bfloat16)

    @pl.when(i > NB)
    def _():
        j = i - NB - 1
        b = batch_ref[pl.ds(j, 1), :]
        g_lo = jnp.min(b)
        g_hi = jnp.max(b)
        g0 = (g_lo // GC) * GC
        gids = jax.lax.broadcasted_iota(jnp.int32, (W, B), 0)
        dn = (((0,), (0,)), ((), ()))
        oh_bf = (gids == jnp.broadcast_to(b - g0, (W, B))).astype(jnp.bfloat16)
        g = jax.lax.dot_general(oh_bf, vn_hi_sc[pl.ds(g0, W), :], dn,
                                preferred_element_type=jnp.float32)
        out_ref[...] = h_sc[pl.ds(j * B, B), :].astype(jnp.float32) + g
        gids_t = jax.lax.broadcasted_iota(jnp.int32, (GC, B), 0)
        for k in range(2, nk):
            @pl.when((g_hi >= k * GC) & (k * GC >= g0 + W))
            def _(k=k):
                oh_bk = (gids_t == jnp.broadcast_to(b - (k * GC), (GC, B))
                         ).astype(jnp.bfloat16)
                gk = jax.lax.dot_general(
                    oh_bk, vn_hi_sc[pl.ds(k * GC, GC), :], dn,
                    preferred_element_type=jnp.float32)
                out_ref[...] += gk


def _pick_block(n):
    for cand in range(10240, 7, -8):
        if n % cand == 0:
            return cand
    return n


_FUSED = True


def kernel(h, batch, vn_h, W1, b1, gamma, beta, W2, b2, layer_idx):
    del layer_idx
    N, D = h.shape
    G = vn_h.shape[0]

    if _FUSED:
        B = 5000 if N % 5000 == 0 else _pick_block(N)
        NB = N // B
        batch2 = batch.astype(jnp.int32).reshape(NB, B)
        h_out, vn_out = pl.pallas_call(
            functools.partial(_fused_body, G=G, B=B, NB=NB, GC=128),
            grid=(2 * NB + 1,),
            in_specs=[
                pl.BlockSpec((NB, B), lambda i: (0, 0)),
                pl.BlockSpec((B, D), lambda i: (jnp.minimum(i, NB - 1), 0)),
                pl.BlockSpec((G, D), lambda i: (0, 0)),
                pl.BlockSpec((D, D), lambda i: (0, 0)),
                pl.BlockSpec((1, D), lambda i: (0, 0)),
                pl.BlockSpec((1, D), lambda i: (0, 0)),
                pl.BlockSpec((1, D), lambda i: (0, 0)),
                pl.BlockSpec((D, D), lambda i: (0, 0)),
                pl.BlockSpec((1, D), lambda i: (0, 0)),
            ],
            out_specs=[
                pl.BlockSpec((B, D),
                             lambda i: (jnp.maximum(i - NB - 1, 0), 0)),
                pl.BlockSpec((G, D), lambda i: (0, 0)),
            ],
            out_shape=[
                jax.ShapeDtypeStruct((N, D), jnp.float32),
                jax.ShapeDtypeStruct((G, D), jnp.float32),
            ],
            scratch_shapes=[
                pltpu.VMEM((N, D), jnp.bfloat16),
                pltpu.VMEM((G + 128, D), jnp.float32),
                pltpu.VMEM((G + 128, 1), jnp.float32),
                pltpu.VMEM((G + 128, D), jnp.bfloat16),
            ],
        )(batch2, h, vn_h, W1, b1.reshape(1, D), gamma.reshape(1, D),
          beta.reshape(1, D), W2, b2.reshape(1, D))
        return (h_out, vn_out)

    B = _pick_block(N)
    NB = N // B
    batch_i = batch.astype(jnp.int32)
    batch3 = batch_i.reshape(NB, 1, B)

    mlp_args = (vn_h, W1, b1.reshape(1, D), gamma.reshape(1, D),
                beta.reshape(1, D), W2, b2.reshape(1, D))
    vn_shapes = [
        jax.ShapeDtypeStruct((G, D), jnp.float32),
        jax.ShapeDtypeStruct((G, D), jnp.bfloat16),
    ]

    if _USE_SC_COUNTS:
        hist = _sc_counts(batch_i, _P)
        sums = pl.pallas_call(
            functools.partial(_phase_a_body_nocnt, G=G),
            grid=(NB,),
            in_specs=[
                pl.BlockSpec((1, 1, B), lambda i: (i, 0, 0)),
                pl.BlockSpec((B, D), lambda i: (i, 0)),
            ],
            out_specs=pl.BlockSpec((G, D), lambda i: (0, 0)),
            out_shape=jax.ShapeDtypeStruct((G, D), jnp.float32),
        )(batch3, h)
        vn_out, vn_hi = pl.pallas_call(
            functools.partial(_phase_b_body_schist, G=G),
            out_shape=vn_shapes,
        )(sums, hist, *mlp_args)
    else:
        sums, counts = pl.pallas_call(
            functools.partial(_phase_a_body, G=G),
            grid=(NB,),
            in_specs=[
                pl.BlockSpec((1, 1, B), lambda i: (i, 0, 0)),
                pl.BlockSpec((B, D), lambda i: (i, 0)),
            ],
            out_specs=[
                pl.BlockSpec((G, D), lambda i: (0, 0)),
                pl.BlockSpec((G, 1), lambda i: (0, 0)),
            ],
            out_shape=[
                jax.ShapeDtypeStruct((G, D), jnp.float32),
                jax.ShapeDtypeStruct((G, 1), jnp.float32),
            ],
        )(batch3, h)
        vn_out, vn_hi = pl.pallas_call(
            _phase_b_body,
            out_shape=vn_shapes,
        )(sums, counts, *mlp_args)

    h_out = pl.pallas_call(
        functools.partial(_phase_c_body, G=G),
        grid=(NB,),
        in_specs=[
            pl.BlockSpec((1, 1, B), lambda i: (i, 0, 0)),
            pl.BlockSpec((B, D), lambda i: (i, 0)),
            pl.BlockSpec((G, D), lambda i: (0, 0)),
        ],
        out_specs=pl.BlockSpec((B, D), lambda i: (i, 0)),
        out_shape=jax.ShapeDtypeStruct((N, D), jnp.float32),
    )(batch3, h, vn_hi)

    return (h_out, vn_out)
